# Initial kernel scaffold; baseline (speedup 1.0000x reference)
import jax
import jax.numpy as jnp
from jax.experimental import pallas as pl


def kernel(x, edge_index, edge_attr, batch, mask, node_W, node_b, edge_W, edge_b, msg_W1_0, msg_b1_0, msg_W2_0, msg_b2_0, upd_W_0, upd_b_0, msg_W1_1, msg_b1_1, msg_W2_1, msg_b2_1, upd_W_1, upd_b_1, msg_W1_2, msg_b1_2, msg_W2_2, msg_b2_2, upd_W_2, upd_b_2, pat_W1, pat_b1, pat_W2, pat_b2, pat_W3, pat_b3, clf_W1, clf_b1, clf_W2, clf_b2):
    raise NotImplementedError("write your pallas kernel here")


# speedup vs baseline: 3.8078x; 3.8078x over previous
import functools

import jax
import jax.numpy as jnp
from jax import lax
from jax.experimental import pallas as pl
from jax.experimental.pallas import tpu as pltpu
from jax.experimental.pallas import tpu_sc as plsc

_NC = 2
_NS = 16
_LN = 16


def _prep_body(x_ref, nW_ref, nb_ref, mW1_ref, mb1_ref, eW_ref, eb_ref,
               h_ref, A_ref, v_ref):
    H = nW_ref.shape[1]
    h = jnp.dot(x_ref[...], nW_ref[...],
                preferred_element_type=jnp.float32) + nb_ref[...]
    W1a = mW1_ref[:H, :]
    W1b = mW1_ref[H:, :]
    c = jnp.dot(eb_ref[...], W1b,
                preferred_element_type=jnp.float32) + mb1_ref[...]
    h_ref[...] = h
    A_ref[...] = jnp.dot(h, W1a, preferred_element_type=jnp.float32) + c
    v_ref[...] = jnp.dot(eW_ref[...], W1b, preferred_element_type=jnp.float32)


def _agg_update(part, h, mW2, mb2, uW, ub):
    H = h.shape[1]
    T = part[0, :, :H] + part[1, :, :H]
    cnt = part[0, :, H:H + 1] + part[1, :, H:H + 1]
    seg = T / jnp.maximum(cnt, 1.0)
    aggr = jnp.dot(seg, mW2, preferred_element_type=jnp.float32) \
        + mb2 * (cnt > 0).astype(jnp.float32)
    hn = jnp.dot(h, uW[:H, :], preferred_element_type=jnp.float32) \
        + jnp.dot(aggr, uW[H:, :], preferred_element_type=jnp.float32) \
        + ub
    return jnp.maximum(hn, 0.0)


def _upd_body(part_ref, h_ref, mW2_ref, mb2_ref, uW_ref, ub_ref,
              mW1n_ref, mb1n_ref, eW_ref, eb_ref,
              hn_ref, An_ref, vn_ref):
    H = h_ref.shape[1]
    hn = _agg_update(part_ref[...], h_ref[...], mW2_ref[...], mb2_ref[...],
                     uW_ref[...], ub_ref[...])
    hn_ref[...] = hn
    W1a = mW1n_ref[:H, :]
    W1b = mW1n_ref[H:, :]
    c = jnp.dot(eb_ref[...], W1b,
                preferred_element_type=jnp.float32) + mb1n_ref[...]
    An_ref[...] = jnp.dot(hn, W1a, preferred_element_type=jnp.float32) + c
    vn_ref[...] = jnp.dot(eW_ref[...], W1b, preferred_element_type=jnp.float32)


def _upd_last_body(part_ref, h_ref, mW2_ref, mb2_ref, uW_ref, ub_ref,
                   hn_ref):
    hn_ref[...] = _agg_update(part_ref[...], h_ref[...], mW2_ref[...],
                              mb2_ref[...], uW_ref[...], ub_ref[...])


def _tail_body(n_real, h_ref, batch_ref, pat_ref,
               pW1_ref, pb1_ref, pW2_ref, pb2_ref, pW3_ref, pb3_ref,
               cW1_ref, cb1_ref, cW2_ref, cb2_ref, out_ref):
    NP = h_ref.shape[0]
    G = pat_ref.shape[0]
    bat = batch_ref[...]
    g = lax.broadcasted_iota(jnp.int32, (G, 1), 0)
    less = (bat < g).astype(jnp.float32)
    fi = jnp.sum(less, axis=1, keepdims=True)
    fi = jnp.minimum(fi, float(n_real - 1)).astype(jnp.int32)
    ii = lax.broadcasted_iota(jnp.int32, (G, NP), 1)
    onehot = (ii == fi).astype(jnp.float32)
    obs = jnp.dot(onehot, h_ref[...], preferred_element_type=jnp.float32)
    p = jnp.maximum(jnp.dot(pat_ref[...], pW1_ref[...],
                            preferred_element_type=jnp.float32) + pb1_ref[...], 0.0)
    p = jnp.maximum(jnp.dot(p, pW2_ref[...],
                            preferred_element_type=jnp.float32) + pb2_ref[...], 0.0)
    p = jnp.dot(p, pW3_ref[...], preferred_element_type=jnp.float32) + pb3_ref[...]
    z = jnp.concatenate([obs, p], axis=1)
    z = jnp.maximum(jnp.dot(z, cW1_ref[...],
                            preferred_element_type=jnp.float32) + cb1_ref[...], 0.0)
    out_ref[...] = jnp.dot(z, cW2_ref[...],
                           preferred_element_type=jnp.float32) + cb2_ref[...]


def _make_edge_sc(Nn, H, E):
    NW = _NC * _NS
    EW = E // NW
    K = 80
    NCH = EW // K
    RT = Nn // _NS
    ZR = 128
    W = H + _LN
    JH = H // _LN

    mesh = plsc.VectorSubcoreMesh(core_axis_name="c", subcore_axis_name="s")

    def body(A_hbm, src_hbm, dst_hbm, attr_hbm, v_hbm, out_hbm,
             src_v, attr_v, dst_b, v_v, rows_v, m_v, zb, S, sem):
        cid = lax.axis_index("c")
        sid = lax.axis_index("s")
        wid = sid * _NC + cid
        base = wid * EW
        pltpu.sync_copy(src_hbm.at[pl.ds(base, EW)], src_v)
        pltpu.sync_copy(attr_hbm.at[pl.ds(base, EW)], attr_v)
        pltpu.sync_copy(v_hbm, v_v)

        z16 = jnp.zeros((_LN,), jnp.float32)
        o16 = jnp.ones((_LN,), jnp.float32)

        def zrow(i, carry):
            for j in range(W // _LN):
                zb[i, pl.ds(j * _LN, _LN)] = z16
            return carry
        lax.fori_loop(0, ZR, zrow, 0)
        row0 = sid * RT
        for kb in range(RT // ZR):
            pltpu.sync_copy(zb, S.at[pl.ds(row0 + kb * ZR, ZR)])

        def orow(i, carry):
            m_v[i, pl.ds(H, _LN)] = o16
            return carry
        lax.fori_loop(0, K, orow, 0)
        plsc.subcore_barrier()

        vjs = [v_v[pl.ds(j * _LN, _LN)] for j in range(JH)]
        lane0 = jnp.zeros((_LN,), jnp.int32)

        def chunk(c, carry):
            off = c * K
            pltpu.sync_copy(dst_hbm.at[pl.ds(base + off, K)], dst_b)
            pltpu.async_copy(A_hbm.at[src_v.at[pl.ds(off, K)]], rows_v,
                             sem).wait()

            def edge(e, carry2):
                a16 = plsc.load_gather(attr_v, [lane0 + (off + e)])
                for j in range(JH):
                    r = rows_v[e, pl.ds(j * _LN, _LN)]
                    m_v[e, pl.ds(j * _LN, _LN)] = jnp.maximum(
                        r + a16 * vjs[j], 0.0)
                return carry2
            lax.fori_loop(0, K, edge, 0)
            pltpu.sync_copy(m_v, S.at[dst_b], add=True)
            return carry
        lax.fori_loop(0, NCH, chunk, 0)

        plsc.subcore_barrier()
        for kb in range(RT // ZR):
            pltpu.sync_copy(S.at[pl.ds(row0 + kb * ZR, ZR)],
                            out_hbm.at[cid, pl.ds(row0 + kb * ZR, ZR)])

    return pl.kernel(
        body,
        out_type=jax.ShapeDtypeStruct((_NC, Nn, W), jnp.float32),
        mesh=mesh,
        compiler_params=pltpu.CompilerParams(needs_layout_passes=False,
                                             use_tc_tiling_on_sc=False),
        scratch_types=[
            pltpu.VMEM((EW,), jnp.int32),
            pltpu.VMEM((EW,), jnp.float32),
            pltpu.VMEM((K,), jnp.int32),
            pltpu.VMEM((H,), jnp.float32),
            pltpu.VMEM((K, H), jnp.float32),
            pltpu.VMEM((K, W), jnp.float32),
            pltpu.VMEM((ZR, W), jnp.float32),
            pltpu.VMEM_SHARED((Nn, W), jnp.float32),
            pltpu.SemaphoreType.DMA,
        ],
    )


def kernel(x, edge_index, edge_attr, batch, mask,
           node_W, node_b, edge_W, edge_b,
           msg_W1_0, msg_b1_0, msg_W2_0, msg_b2_0, upd_W_0, upd_b_0,
           msg_W1_1, msg_b1_1, msg_W2_1, msg_b2_1, upd_W_1, upd_b_1,
           msg_W1_2, msg_b1_2, msg_W2_2, msg_b2_2, upd_W_2, upd_b_2,
           pat_W1, pat_b1, pat_W2, pat_b2, pat_W3, pat_b3,
           clf_W1, clf_b1, clf_W2, clf_b2):
    N, F = x.shape
    H = node_W.shape[1]
    E = edge_index.shape[1]
    G = mask.shape[0] // F
    C = clf_W2.shape[1]
    f32 = jnp.float32
    r1 = lambda b: b.reshape(1, -1)

    ALIGN = _NS * 128
    NP = ((N + ALIGN - 1) // ALIGN) * ALIGN
    src = edge_index[0]
    dst = edge_index[1]
    attr = edge_attr.reshape(E)
    xp = jnp.pad(x, ((0, NP - N), (0, 0)))
    batch_p = jnp.pad(batch, (0, NP - N), constant_values=jnp.int32(2**30))

    h, A, v = pl.pallas_call(
        _prep_body,
        out_shape=(jax.ShapeDtypeStruct((NP, H), f32),
                   jax.ShapeDtypeStruct((NP, H), f32),
                   jax.ShapeDtypeStruct((1, H), f32)),
    )(xp, node_W, r1(node_b), msg_W1_0, r1(msg_b1_0), edge_W, r1(edge_b))

    edge_sc = _make_edge_sc(NP, H, E)

    upd = pl.pallas_call(
        _upd_body,
        out_shape=(jax.ShapeDtypeStruct((NP, H), f32),
                   jax.ShapeDtypeStruct((NP, H), f32),
                   jax.ShapeDtypeStruct((1, H), f32)),
    )
    upd_last = pl.pallas_call(
        _upd_last_body,
        out_shape=jax.ShapeDtypeStruct((NP, H), f32),
    )

    layers = [
        (msg_W1_0, msg_b1_0, msg_W2_0, msg_b2_0, upd_W_0, upd_b_0),
        (msg_W1_1, msg_b1_1, msg_W2_1, msg_b2_1, upd_W_1, upd_b_1),
        (msg_W1_2, msg_b1_2, msg_W2_2, msg_b2_2, upd_W_2, upd_b_2),
    ]
    for l, (_, _, mW2, mb2, uW, ub) in enumerate(layers):
        part = edge_sc(A, src, dst, attr, v.reshape(H))
        if l < 2:
            mW1n, mb1n = layers[l + 1][0], layers[l + 1][1]
            h, A, v = upd(part, h, mW2, r1(mb2), uW, r1(ub),
                          mW1n, r1(mb1n), edge_W, r1(edge_b))
        else:
            h = upd_last(part, h, mW2, r1(mb2), uW, r1(ub))

    out = pl.pallas_call(
        functools.partial(_tail_body, N),
        out_shape=jax.ShapeDtypeStruct((G, C), f32),
    )(h, batch_p.reshape(1, NP), mask.reshape(G, F),
      pat_W1, r1(pat_b1), pat_W2, r1(pat_b2), pat_W3, r1(pat_b3),
      clf_W1, r1(clf_b1), clf_W2, r1(clf_b2))
    return out


# speedup vs baseline: 6.3106x; 1.6573x over previous
import functools

import jax
import jax.numpy as jnp
from jax import lax
from jax.experimental import pallas as pl
from jax.experimental.pallas import tpu as pltpu
from jax.experimental.pallas import tpu_sc as plsc

_NC = 2
_NS = 16
_LN = 16
_K = 80


def _prep_body(x_ref, nW_ref, nb_ref, mW1_ref, mb1_ref, eW_ref, eb_ref,
               h_ref, A_ref, v_ref):
    H = nW_ref.shape[1]
    h = jnp.dot(x_ref[...], nW_ref[...],
                preferred_element_type=jnp.float32) + nb_ref[...]
    W1a = mW1_ref[:H, :]
    W1b = mW1_ref[H:, :]
    c = jnp.dot(eb_ref[...], W1b,
                preferred_element_type=jnp.float32) + mb1_ref[...]
    h_ref[...] = h
    A_ref[...] = jnp.dot(h, W1a, preferred_element_type=jnp.float32) + c
    v_ref[...] = jnp.dot(eW_ref[...], W1b, preferred_element_type=jnp.float32)


def _agg_update(part, h, mW2, mb2, uW, ub):
    H = h.shape[1]
    T = part[0, :, :H] + part[1, :, :H]
    cnt = part[0, :, H:H + 1] + part[1, :, H:H + 1]
    seg = T / jnp.maximum(cnt, 1.0)
    aggr = jnp.dot(seg, mW2, preferred_element_type=jnp.float32) \
        + mb2 * (cnt > 0).astype(jnp.float32)
    hn = jnp.dot(h, uW[:H, :], preferred_element_type=jnp.float32) \
        + jnp.dot(aggr, uW[H:, :], preferred_element_type=jnp.float32) \
        + ub
    return jnp.maximum(hn, 0.0)


def _upd_body(part_ref, h_ref, mW2_ref, mb2_ref, uW_ref, ub_ref,
              mW1n_ref, mb1n_ref, eW_ref, eb_ref,
              hn_ref, An_ref, vn_ref):
    H = h_ref.shape[1]
    hn = _agg_update(part_ref[...], h_ref[...], mW2_ref[...], mb2_ref[...],
                     uW_ref[...], ub_ref[...])
    hn_ref[...] = hn
    W1a = mW1n_ref[:H, :]
    W1b = mW1n_ref[H:, :]
    c = jnp.dot(eb_ref[...], W1b,
                preferred_element_type=jnp.float32) + mb1n_ref[...]
    An_ref[...] = jnp.dot(hn, W1a, preferred_element_type=jnp.float32) + c
    vn_ref[...] = jnp.dot(eW_ref[...], W1b, preferred_element_type=jnp.float32)


def _upd_last_body(part_ref, h_ref, mW2_ref, mb2_ref, uW_ref, ub_ref,
                   hn_ref):
    hn_ref[...] = _agg_update(part_ref[...], h_ref[...], mW2_ref[...],
                              mb2_ref[...], uW_ref[...], ub_ref[...])


def _tail_body(n_real, h_ref, batch_ref, pat_ref,
               pW1_ref, pb1_ref, pW2_ref, pb2_ref, pW3_ref, pb3_ref,
               cW1_ref, cb1_ref, cW2_ref, cb2_ref, out_ref):
    NP = h_ref.shape[0]
    G = pat_ref.shape[0]
    bat = batch_ref[...]
    g = lax.broadcasted_iota(jnp.int32, (G, 1), 0)
    less = (bat < g).astype(jnp.float32)
    fi = jnp.sum(less, axis=1, keepdims=True)
    fi = jnp.minimum(fi, float(n_real - 1)).astype(jnp.int32)
    ii = lax.broadcasted_iota(jnp.int32, (G, NP), 1)
    onehot = (ii == fi).astype(jnp.float32)
    obs = jnp.dot(onehot, h_ref[...], preferred_element_type=jnp.float32)
    p = jnp.maximum(jnp.dot(pat_ref[...], pW1_ref[...],
                            preferred_element_type=jnp.float32) + pb1_ref[...], 0.0)
    p = jnp.maximum(jnp.dot(p, pW2_ref[...],
                            preferred_element_type=jnp.float32) + pb2_ref[...], 0.0)
    p = jnp.dot(p, pW3_ref[...], preferred_element_type=jnp.float32) + pb3_ref[...]
    z = jnp.concatenate([obs, p], axis=1)
    z = jnp.maximum(jnp.dot(z, cW1_ref[...],
                            preferred_element_type=jnp.float32) + cb1_ref[...], 0.0)
    out_ref[...] = jnp.dot(z, cW2_ref[...],
                           preferred_element_type=jnp.float32) + cb2_ref[...]


def _make_edge_sc(Nn, H, E):
    NW = _NC * _NS
    EW = E // NW
    K = _K
    NCH = EW // K
    RT = Nn // _NS
    ZR = 128
    W = H + _LN
    JH = H // _LN

    mesh = plsc.VectorSubcoreMesh(core_axis_name="c", subcore_axis_name="s")

    def body(A_hbm, src_hbm, dst_hbm, attr_hbm, v_hbm, out_hbm,
             src_v, attr_v, dst_vm, v_v, rows0, rows1, m0, m1, zb, S,
             gs0, gs1, ss0, ss1):
        cid = lax.axis_index("c")
        sid = lax.axis_index("s")
        wid = sid * _NC + cid
        base = wid * EW
        pltpu.sync_copy(src_hbm.at[pl.ds(base, EW)], src_v)
        pltpu.sync_copy(attr_hbm.at[pl.ds(base, EW)], attr_v)
        pltpu.sync_copy(dst_hbm.at[wid], dst_vm)
        pltpu.sync_copy(v_hbm, v_v)

        z16 = jnp.zeros((_LN,), jnp.float32)
        o16 = jnp.ones((_LN,), jnp.float32)

        def zrow(i, carry):
            for j in range(W // _LN):
                zb[i, pl.ds(j * _LN, _LN)] = z16
            return carry
        lax.fori_loop(0, ZR, zrow, 0)
        row0_ = sid * RT
        for kb in range(RT // ZR):
            pltpu.sync_copy(zb, S.at[pl.ds(row0_ + kb * ZR, ZR)])

        def orow(i, carry):
            m0[i, pl.ds(H, _LN)] = o16
            m1[i, pl.ds(H, _LN)] = o16
            return carry
        lax.fori_loop(0, K, orow, 0)
        plsc.subcore_barrier()

        vjs = [v_v[pl.ds(j * _LN, _LN)] for j in range(JH)]
        lane0 = jnp.zeros((_LN,), jnp.int32)

        def g_start(c, buf, sem):
            pltpu.async_copy(A_hbm.at[src_v.at[pl.ds(c * K, K)]], buf, sem)

        def g_wait(c, buf, sem):
            pltpu.make_async_copy(
                A_hbm.at[src_v.at[pl.ds(c * K, K)]], buf, sem).wait()

        def s_start(m, c, sem):
            pltpu.async_copy(m, S.at[dst_vm.at[c]], sem, add=True)

        def s_wait(m, c, sem):
            pltpu.make_async_copy(m, S.at[dst_vm.at[c]], sem).wait()

        def compute(c, rows, m):
            base16 = lane0 + c * K

            def edge(e, carry):
                a16 = plsc.load_gather(attr_v, [base16 + e])
                for j in range(JH):
                    m[e, pl.ds(j * _LN, _LN)] = jnp.maximum(
                        rows[e, pl.ds(j * _LN, _LN)] + a16 * vjs[j], 0.0)
                return carry
            lax.fori_loop(0, K, edge, 0, unroll=8)

        g_start(0, rows0, gs0)
        g_wait(0, rows0, gs0)
        compute(0, rows0, m0)
        s_start(m0, 0, ss0)
        g_start(1, rows1, gs1)
        g_wait(1, rows1, gs1)
        compute(1, rows1, m1)
        s_start(m1, 1, ss1)
        g_start(2, rows0, gs0)
        g_start(3, rows1, gs1)

        last = NCH - 1

        def pair(i, carry):
            c0 = 2 * i
            g_wait(c0, rows0, gs0)
            s_wait(m0, c0, ss0)
            compute(c0, rows0, m0)
            s_start(m0, c0, ss0)
            g_start(jnp.minimum(c0 + 2, last), rows0, gs0)
            c1 = c0 + 1
            g_wait(c1, rows1, gs1)
            s_wait(m1, c1, ss1)
            compute(c1, rows1, m1)
            s_start(m1, c1, ss1)
            g_start(jnp.minimum(c1 + 2, last), rows1, gs1)
            return carry
        lax.fori_loop(1, (NCH - 1) // 2, pair, 0)

        g_wait(last, rows0, gs0)
        s_wait(m0, last, ss0)
        compute(last, rows0, m0)
        s_start(m0, last, ss0)
        g_wait(last, rows1, gs1)
        s_wait(m1, last, ss1)
        s_wait(m0, last, ss0)

        plsc.subcore_barrier()
        for kb in range(RT // ZR):
            pltpu.sync_copy(S.at[pl.ds(row0_ + kb * ZR, ZR)],
                            out_hbm.at[cid, pl.ds(row0_ + kb * ZR, ZR)])

    return pl.kernel(
        body,
        out_type=jax.ShapeDtypeStruct((_NC, Nn, W), jnp.float32),
        mesh=mesh,
        compiler_params=pltpu.CompilerParams(needs_layout_passes=False,
                                             use_tc_tiling_on_sc=False),
        scratch_types=[
            pltpu.VMEM((EW,), jnp.int32),
            pltpu.VMEM((EW,), jnp.float32),
            pltpu.VMEM((NCH, K), jnp.int32),
            pltpu.VMEM((H,), jnp.float32),
            pltpu.VMEM((K, H), jnp.float32),
            pltpu.VMEM((K, H), jnp.float32),
            pltpu.VMEM((K, W), jnp.float32),
            pltpu.VMEM((K, W), jnp.float32),
            pltpu.VMEM((ZR, W), jnp.float32),
            pltpu.VMEM_SHARED((Nn, W), jnp.float32),
            pltpu.SemaphoreType.DMA,
            pltpu.SemaphoreType.DMA,
            pltpu.SemaphoreType.DMA,
            pltpu.SemaphoreType.DMA,
        ],
    )


def kernel(x, edge_index, edge_attr, batch, mask,
           node_W, node_b, edge_W, edge_b,
           msg_W1_0, msg_b1_0, msg_W2_0, msg_b2_0, upd_W_0, upd_b_0,
           msg_W1_1, msg_b1_1, msg_W2_1, msg_b2_1, upd_W_1, upd_b_1,
           msg_W1_2, msg_b1_2, msg_W2_2, msg_b2_2, upd_W_2, upd_b_2,
           pat_W1, pat_b1, pat_W2, pat_b2, pat_W3, pat_b3,
           clf_W1, clf_b1, clf_W2, clf_b2):
    N, F = x.shape
    H = node_W.shape[1]
    E = edge_index.shape[1]
    G = mask.shape[0] // F
    C = clf_W2.shape[1]
    f32 = jnp.float32
    r1 = lambda b: b.reshape(1, -1)

    ALIGN = _NS * 128
    NP = ((N + ALIGN - 1) // ALIGN) * ALIGN
    src = edge_index[0]
    dst = edge_index[1].reshape(_NC * _NS, E // (_NC * _NS) // _K, _K)
    attr = edge_attr.reshape(E)
    xp = jnp.pad(x, ((0, NP - N), (0, 0)))
    batch_p = jnp.pad(batch, (0, NP - N), constant_values=jnp.int32(2**30))

    h, A, v = pl.pallas_call(
        _prep_body,
        out_shape=(jax.ShapeDtypeStruct((NP, H), f32),
                   jax.ShapeDtypeStruct((NP, H), f32),
                   jax.ShapeDtypeStruct((1, H), f32)),
    )(xp, node_W, r1(node_b), msg_W1_0, r1(msg_b1_0), edge_W, r1(edge_b))

    edge_sc = _make_edge_sc(NP, H, E)

    upd = pl.pallas_call(
        _upd_body,
        out_shape=(jax.ShapeDtypeStruct((NP, H), f32),
                   jax.ShapeDtypeStruct((NP, H), f32),
                   jax.ShapeDtypeStruct((1, H), f32)),
    )
    upd_last = pl.pallas_call(
        _upd_last_body,
        out_shape=jax.ShapeDtypeStruct((NP, H), f32),
    )

    layers = [
        (msg_W1_0, msg_b1_0, msg_W2_0, msg_b2_0, upd_W_0, upd_b_0),
        (msg_W1_1, msg_b1_1, msg_W2_1, msg_b2_1, upd_W_1, upd_b_1),
        (msg_W1_2, msg_b1_2, msg_W2_2, msg_b2_2, upd_W_2, upd_b_2),
    ]
    for l, (_, _, mW2, mb2, uW, ub) in enumerate(layers):
        part = edge_sc(A, src, dst, attr, v.reshape(H))
        if l < 2:
            mW1n, mb1n = layers[l + 1][0], layers[l + 1][1]
            h, A, v = upd(part, h, mW2, r1(mb2), uW, r1(ub),
                          mW1n, r1(mb1n), edge_W, r1(edge_b))
        else:
            h = upd_last(part, h, mW2, r1(mb2), uW, r1(ub))

    out = pl.pallas_call(
        functools.partial(_tail_body, N),
        out_shape=jax.ShapeDtypeStruct((G, C), f32),
    )(h, batch_p.reshape(1, NP), mask.reshape(G, F),
      pat_W1, r1(pat_b1), pat_W2, r1(pat_b2), pat_W3, r1(pat_b3),
      clf_W1, r1(clf_b1), clf_W2, r1(clf_b2))
    return out


# speedup vs baseline: 6.3255x; 1.0024x over previous
import functools

import jax
import jax.numpy as jnp
from jax import lax
from jax.experimental import pallas as pl
from jax.experimental.pallas import tpu as pltpu
from jax.experimental.pallas import tpu_sc as plsc

_NC = 2
_NS = 16
_LN = 16
_K = 80


def _prep_body(x_ref, nW_ref, nb_ref, mW1_ref, mb1_ref, eW_ref, eb_ref,
               h_ref, A_ref, v_ref):
    H = nW_ref.shape[1]
    h = jnp.dot(x_ref[...], nW_ref[...],
                preferred_element_type=jnp.float32) + nb_ref[...]
    W1a = mW1_ref[:H, :]
    W1b = mW1_ref[H:, :]
    c = jnp.dot(eb_ref[...], W1b,
                preferred_element_type=jnp.float32) + mb1_ref[...]
    h_ref[...] = h
    A_ref[...] = jnp.dot(h, W1a, preferred_element_type=jnp.float32) + c
    v_ref[...] = jnp.dot(eW_ref[...], W1b, preferred_element_type=jnp.float32)


def _agg_update(T, cnt, h, mW2, mb2, uW, ub):
    H = h.shape[1]
    seg = T / jnp.maximum(cnt, 1.0)
    aggr = jnp.dot(seg, mW2, preferred_element_type=jnp.float32) \
        + mb2 * (cnt > 0).astype(jnp.float32)
    hn = jnp.dot(h, uW[:H, :], preferred_element_type=jnp.float32) \
        + jnp.dot(aggr, uW[H:, :], preferred_element_type=jnp.float32) \
        + ub
    return jnp.maximum(hn, 0.0)


def _next_A(hn, mW1n, mb1n, eW, eb):
    H = hn.shape[1]
    W1a = mW1n[:H, :]
    W1b = mW1n[H:, :]
    c = jnp.dot(eb, W1b, preferred_element_type=jnp.float32) + mb1n
    An = jnp.dot(hn, W1a, preferred_element_type=jnp.float32) + c
    vn = jnp.dot(eW, W1b, preferred_element_type=jnp.float32)
    return An, vn


def _upd0_body(part_ref, ch_ref, h_ref, mW2_ref, mb2_ref, uW_ref, ub_ref,
               mW1n_ref, mb1n_ref, eW_ref, eb_ref,
               hn_ref, An_ref, vn_ref, cnt_ref):
    p = part_ref[...]
    T = p[0] + p[1]
    cnt = jnp.sum(ch_ref[...], axis=(0, 1))[:, None]
    cnt_ref[...] = cnt
    hn = _agg_update(T, cnt, h_ref[...], mW2_ref[...], mb2_ref[...],
                     uW_ref[...], ub_ref[...])
    hn_ref[...] = hn
    An_ref[...], vn_ref[...] = _next_A(hn, mW1n_ref[...], mb1n_ref[...],
                                       eW_ref[...], eb_ref[...])


def _upd_mid_body(part_ref, cnt_ref, h_ref, mW2_ref, mb2_ref, uW_ref, ub_ref,
                  mW1n_ref, mb1n_ref, eW_ref, eb_ref,
                  hn_ref, An_ref, vn_ref):
    p = part_ref[...]
    T = p[0] + p[1]
    hn = _agg_update(T, cnt_ref[...], h_ref[...], mW2_ref[...], mb2_ref[...],
                     uW_ref[...], ub_ref[...])
    hn_ref[...] = hn
    An_ref[...], vn_ref[...] = _next_A(hn, mW1n_ref[...], mb1n_ref[...],
                                       eW_ref[...], eb_ref[...])


def _upd_last_body(part_ref, cnt_ref, h_ref, mW2_ref, mb2_ref, uW_ref, ub_ref,
                   hn_ref):
    p = part_ref[...]
    T = p[0] + p[1]
    hn_ref[...] = _agg_update(T, cnt_ref[...], h_ref[...], mW2_ref[...],
                              mb2_ref[...], uW_ref[...], ub_ref[...])


def _tail_body(n_real, h_ref, batch_ref, pat_ref,
               pW1_ref, pb1_ref, pW2_ref, pb2_ref, pW3_ref, pb3_ref,
               cW1_ref, cb1_ref, cW2_ref, cb2_ref, out_ref):
    NP = h_ref.shape[0]
    G = pat_ref.shape[0]
    bat = batch_ref[...]
    g = lax.broadcasted_iota(jnp.int32, (G, 1), 0)
    less = (bat < g).astype(jnp.float32)
    fi = jnp.sum(less, axis=1, keepdims=True)
    fi = jnp.minimum(fi, float(n_real - 1)).astype(jnp.int32)
    ii = lax.broadcasted_iota(jnp.int32, (G, NP), 1)
    onehot = (ii == fi).astype(jnp.float32)
    obs = jnp.dot(onehot, h_ref[...], preferred_element_type=jnp.float32)
    p = jnp.maximum(jnp.dot(pat_ref[...], pW1_ref[...],
                            preferred_element_type=jnp.float32) + pb1_ref[...], 0.0)
    p = jnp.maximum(jnp.dot(p, pW2_ref[...],
                            preferred_element_type=jnp.float32) + pb2_ref[...], 0.0)
    p = jnp.dot(p, pW3_ref[...], preferred_element_type=jnp.float32) + pb3_ref[...]
    z = jnp.concatenate([obs, p], axis=1)
    z = jnp.maximum(jnp.dot(z, cW1_ref[...],
                            preferred_element_type=jnp.float32) + cb1_ref[...], 0.0)
    out_ref[...] = jnp.dot(z, cW2_ref[...],
                           preferred_element_type=jnp.float32) + cb2_ref[...]


def _make_edge_sc(Nn, H, E, with_cnt):
    NW = _NC * _NS
    EW = E // NW
    K = _K
    NCH = EW // K
    RT = Nn // _NS
    ZR = 128
    W = H + _LN if with_cnt else H
    JH = H // _LN

    mesh = plsc.VectorSubcoreMesh(core_axis_name="c", subcore_axis_name="s")

    D = 4

    def body(A_hbm, src_hbm, dst_hbm, attr_hbm, v_hbm, out_hbm,
             src_v, attr_v, dst_vm, v_v,
             rows0, rows1, rows2, rows3, m0, m1, m2, m3, zb, S,
             gs0, gs1, gs2, gs3, ss0, ss1, ss2, ss3):
        rows = [rows0, rows1, rows2, rows3]
        ms = [m0, m1, m2, m3]
        gs = [gs0, gs1, gs2, gs3]
        ss = [ss0, ss1, ss2, ss3]
        cid = lax.axis_index("c")
        sid = lax.axis_index("s")
        wid = sid * _NC + cid
        base = wid * EW
        pltpu.sync_copy(src_hbm.at[pl.ds(base, EW)], src_v)
        pltpu.sync_copy(attr_hbm.at[pl.ds(base, EW)], attr_v)
        pltpu.sync_copy(dst_hbm.at[wid], dst_vm)
        pltpu.sync_copy(v_hbm, v_v)

        z16 = jnp.zeros((_LN,), jnp.float32)
        o16 = jnp.ones((_LN,), jnp.float32)

        def zrow(i, carry):
            for j in range(W // _LN):
                zb[i, pl.ds(j * _LN, _LN)] = z16
            return carry
        lax.fori_loop(0, ZR, zrow, 0)
        row0_ = sid * RT
        for kb in range(RT // ZR):
            pltpu.sync_copy(zb, S.at[pl.ds(row0_ + kb * ZR, ZR)])

        if with_cnt:
            def orow(i, carry):
                for m in ms:
                    m[i, pl.ds(H, _LN)] = o16
                return carry
            lax.fori_loop(0, K, orow, 0)
        plsc.subcore_barrier()

        vjs = [v_v[pl.ds(j * _LN, _LN)] for j in range(JH)]
        lane0 = jnp.zeros((_LN,), jnp.int32)

        def g_start(c, p):
            pltpu.async_copy(A_hbm.at[src_v.at[pl.ds(c * K, K)]],
                             rows[p], gs[p])

        def g_wait(c, p):
            pltpu.make_async_copy(
                A_hbm.at[src_v.at[pl.ds(c * K, K)]], rows[p], gs[p]).wait()

        def s_start(c, p):
            pltpu.async_copy(ms[p], S.at[dst_vm.at[c]], ss[p], add=True)

        def s_wait(c, p):
            pltpu.make_async_copy(ms[p], S.at[dst_vm.at[c]], ss[p]).wait()

        def compute(c, p):
            base16 = lane0 + c * K
            r, m = rows[p], ms[p]

            def edge(e, carry):
                a16 = plsc.load_gather(attr_v, [base16 + e])
                for j in range(JH):
                    m[e, pl.ds(j * _LN, _LN)] = jnp.maximum(
                        r[e, pl.ds(j * _LN, _LN)] + a16 * vjs[j], 0.0)
                return carry
            lax.fori_loop(0, K, edge, 0, unroll=8)

        def do_chunk(c, p, first, prefetch):
            g_wait(c, p)
            if not first:
                s_wait(c, p)
            compute(c, p)
            s_start(c, p)
            if prefetch is not None:
                g_start(prefetch, p)

        last = NCH - 1
        for p in range(D):
            g_start(p, p)
        for p in range(D):
            do_chunk(jnp.int32(p), p, True, p + D)

        def block(i, carry):
            for p in range(D):
                c = D * i + p
                do_chunk(c, p, False, jnp.minimum(c + D, last))
            return carry
        lax.fori_loop(1, (NCH - 1) // D, block, 0)

        do_chunk(jnp.int32(last), 0, False, None)
        for p in range(1, D):
            g_wait(last, p)
            s_wait(last, p)
        s_wait(last, 0)

        plsc.subcore_barrier()
        for kb in range(RT // ZR):
            pltpu.sync_copy(S.at[pl.ds(row0_ + kb * ZR, ZR)],
                            out_hbm.at[cid, pl.ds(row0_ + kb * ZR, ZR)])

    return pl.kernel(
        body,
        out_type=jax.ShapeDtypeStruct((_NC, Nn, W), jnp.float32),
        mesh=mesh,
        compiler_params=pltpu.CompilerParams(needs_layout_passes=False,
                                             use_tc_tiling_on_sc=False),
        scratch_types=[
            pltpu.VMEM((EW,), jnp.int32),
            pltpu.VMEM((EW,), jnp.float32),
            pltpu.VMEM((NCH, K), jnp.int32),
            pltpu.VMEM((H,), jnp.float32),
            pltpu.VMEM((K, H), jnp.float32),
            pltpu.VMEM((K, H), jnp.float32),
            pltpu.VMEM((K, H), jnp.float32),
            pltpu.VMEM((K, H), jnp.float32),
            pltpu.VMEM((K, W), jnp.float32),
            pltpu.VMEM((K, W), jnp.float32),
            pltpu.VMEM((K, W), jnp.float32),
            pltpu.VMEM((K, W), jnp.float32),
            pltpu.VMEM((ZR, W), jnp.float32),
            pltpu.VMEM_SHARED((Nn, W), jnp.float32),
            pltpu.SemaphoreType.DMA,
            pltpu.SemaphoreType.DMA,
            pltpu.SemaphoreType.DMA,
            pltpu.SemaphoreType.DMA,
            pltpu.SemaphoreType.DMA,
            pltpu.SemaphoreType.DMA,
            pltpu.SemaphoreType.DMA,
            pltpu.SemaphoreType.DMA,
        ],
    )


def _make_cnt_sc(Nn, E):
    NW = _NC * _NS
    EW = E // NW
    mesh = plsc.VectorSubcoreMesh(core_axis_name="c", subcore_axis_name="s")

    def body(dst_hbm, out_hbm, dst_v, hist):
        cid = lax.axis_index("c")
        sid = lax.axis_index("s")
        wid = sid * _NC + cid
        pltpu.sync_copy(dst_hbm.at[pl.ds(wid * EW, EW)], dst_v)
        z16 = jnp.zeros((_LN,), jnp.float32)
        o16 = jnp.ones((_LN,), jnp.float32)

        def zrow(i, carry):
            hist[pl.ds(i * _LN, _LN)] = z16
            return carry
        lax.fori_loop(0, Nn // _LN, zrow, 0, unroll=8)

        def step(i, carry):
            idx = dst_v[pl.ds(i * _LN, _LN)]
            plsc.addupdate_scatter(hist, [idx], o16)
            return carry
        lax.fori_loop(0, EW // _LN, step, 0, unroll=8)
        pltpu.sync_copy(hist, out_hbm.at[cid, sid])

    return pl.kernel(
        body,
        out_type=jax.ShapeDtypeStruct((_NC, _NS, Nn), jnp.float32),
        mesh=mesh,
        compiler_params=pltpu.CompilerParams(needs_layout_passes=False,
                                             use_tc_tiling_on_sc=False),
        scratch_types=[
            pltpu.VMEM((EW,), jnp.int32),
            pltpu.VMEM((Nn,), jnp.float32),
        ],
    )


def kernel(x, edge_index, edge_attr, batch, mask,
           node_W, node_b, edge_W, edge_b,
           msg_W1_0, msg_b1_0, msg_W2_0, msg_b2_0, upd_W_0, upd_b_0,
           msg_W1_1, msg_b1_1, msg_W2_1, msg_b2_1, upd_W_1, upd_b_1,
           msg_W1_2, msg_b1_2, msg_W2_2, msg_b2_2, upd_W_2, upd_b_2,
           pat_W1, pat_b1, pat_W2, pat_b2, pat_W3, pat_b3,
           clf_W1, clf_b1, clf_W2, clf_b2):
    N, F = x.shape
    H = node_W.shape[1]
    E = edge_index.shape[1]
    G = mask.shape[0] // F
    C = clf_W2.shape[1]
    f32 = jnp.float32
    r1 = lambda b: b.reshape(1, -1)

    ALIGN = _NS * 128
    NP = ((N + ALIGN - 1) // ALIGN) * ALIGN
    src = edge_index[0]
    dst = edge_index[1].reshape(_NC * _NS, E // (_NC * _NS) // _K, _K)
    attr = edge_attr.reshape(E)
    xp = jnp.pad(x, ((0, NP - N), (0, 0)))
    batch_p = jnp.pad(batch, (0, NP - N), constant_values=jnp.int32(2**30))

    h, A, v = pl.pallas_call(
        _prep_body,
        out_shape=(jax.ShapeDtypeStruct((NP, H), f32),
                   jax.ShapeDtypeStruct((NP, H), f32),
                   jax.ShapeDtypeStruct((1, H), f32)),
    )(xp, node_W, r1(node_b), msg_W1_0, r1(msg_b1_0), edge_W, r1(edge_b))

    edge_sc = _make_edge_sc(NP, H, E, False)
    cnt_sc = _make_cnt_sc(NP, E)

    upd0 = pl.pallas_call(
        _upd0_body,
        out_shape=(jax.ShapeDtypeStruct((NP, H), f32),
                   jax.ShapeDtypeStruct((NP, H), f32),
                   jax.ShapeDtypeStruct((1, H), f32),
                   jax.ShapeDtypeStruct((NP, 1), f32)),
    )
    upd_mid = pl.pallas_call(
        _upd_mid_body,
        out_shape=(jax.ShapeDtypeStruct((NP, H), f32),
                   jax.ShapeDtypeStruct((NP, H), f32),
                   jax.ShapeDtypeStruct((1, H), f32)),
    )
    upd_last = pl.pallas_call(
        _upd_last_body,
        out_shape=jax.ShapeDtypeStruct((NP, H), f32),
    )

    cnthist = cnt_sc(edge_index[1])
    part = edge_sc(A, src, dst, attr, v.reshape(H))
    h, A, v, cnt = upd0(part, cnthist, h, msg_W2_0, r1(msg_b2_0),
                        upd_W_0, r1(upd_b_0),
                        msg_W1_1, r1(msg_b1_1), edge_W, r1(edge_b))
    part = edge_sc(A, src, dst, attr, v.reshape(H))
    h, A, v = upd_mid(part, cnt, h, msg_W2_1, r1(msg_b2_1),
                      upd_W_1, r1(upd_b_1),
                      msg_W1_2, r1(msg_b1_2), edge_W, r1(edge_b))
    part = edge_sc(A, src, dst, attr, v.reshape(H))
    h = upd_last(part, cnt, h, msg_W2_2, r1(msg_b2_2), upd_W_2, r1(upd_b_2))

    out = pl.pallas_call(
        functools.partial(_tail_body, N),
        out_shape=jax.ShapeDtypeStruct((G, C), f32),
    )(h, batch_p.reshape(1, NP), mask.reshape(G, F),
      pat_W1, r1(pat_b1), pat_W2, r1(pat_b2), pat_W3, r1(pat_b3),
      clf_W1, r1(clf_b1), clf_W2, r1(clf_b2))
    return out


# speedup vs baseline: 15.5022x; 2.4508x over previous
import functools

import jax
import jax.numpy as jnp
from jax import lax
from jax.experimental import pallas as pl
from jax.experimental.pallas import tpu as pltpu
from jax.experimental.pallas import tpu_sc as plsc

_NC = 2
_NS = 16
_LN = 16
_K = 80


def _prep_body(x_ref, nW_ref, nb_ref, mW1_ref, mb1_ref, eW_ref, eb_ref,
               h_ref, A_ref, v_ref):
    H = nW_ref.shape[1]
    h = jnp.dot(x_ref[...], nW_ref[...],
                preferred_element_type=jnp.float32) + nb_ref[...]
    W1a = mW1_ref[:H, :]
    W1b = mW1_ref[H:, :]
    c = jnp.dot(eb_ref[...], W1b,
                preferred_element_type=jnp.float32) + mb1_ref[...]
    h_ref[...] = h
    A_ref[...] = jnp.dot(h, W1a, preferred_element_type=jnp.float32) + c
    v_ref[...] = jnp.dot(eW_ref[...], W1b, preferred_element_type=jnp.float32)


def _agg_update(T, cnt, h, mW2, mb2, uW, ub):
    H = h.shape[1]
    seg = T / jnp.maximum(cnt, 1.0)
    aggr = jnp.dot(seg, mW2, preferred_element_type=jnp.float32) \
        + mb2 * (cnt > 0).astype(jnp.float32)
    hn = jnp.dot(h, uW[:H, :], preferred_element_type=jnp.float32) \
        + jnp.dot(aggr, uW[H:, :], preferred_element_type=jnp.float32) \
        + ub
    return jnp.maximum(hn, 0.0)


def _next_A(hn, mW1n, mb1n, eW, eb):
    H = hn.shape[1]
    W1a = mW1n[:H, :]
    W1b = mW1n[H:, :]
    c = jnp.dot(eb, W1b, preferred_element_type=jnp.float32) + mb1n
    An = jnp.dot(hn, W1a, preferred_element_type=jnp.float32) + c
    vn = jnp.dot(eW, W1b, preferred_element_type=jnp.float32)
    return An, vn


def _upd0_body(part_ref, ch_ref, h_ref, mW2_ref, mb2_ref, uW_ref, ub_ref,
               mW1n_ref, mb1n_ref, eW_ref, eb_ref,
               hn_ref, An_ref, vn_ref, cnt_ref):
    p = part_ref[...]
    T = p[0] + p[1]
    cnt = jnp.sum(ch_ref[...], axis=(0, 1))[:, None]
    cnt_ref[...] = cnt
    hn = _agg_update(T, cnt, h_ref[...], mW2_ref[...], mb2_ref[...],
                     uW_ref[...], ub_ref[...])
    hn_ref[...] = hn
    An_ref[...], vn_ref[...] = _next_A(hn, mW1n_ref[...], mb1n_ref[...],
                                       eW_ref[...], eb_ref[...])


def _upd_mid_body(part_ref, cnt_ref, h_ref, mW2_ref, mb2_ref, uW_ref, ub_ref,
                  mW1n_ref, mb1n_ref, eW_ref, eb_ref,
                  hn_ref, An_ref, vn_ref):
    p = part_ref[...]
    T = p[0] + p[1]
    hn = _agg_update(T, cnt_ref[...], h_ref[...], mW2_ref[...], mb2_ref[...],
                     uW_ref[...], ub_ref[...])
    hn_ref[...] = hn
    An_ref[...], vn_ref[...] = _next_A(hn, mW1n_ref[...], mb1n_ref[...],
                                       eW_ref[...], eb_ref[...])


def _upd_last_body(part_ref, cnt_ref, h_ref, mW2_ref, mb2_ref, uW_ref, ub_ref,
                   hn_ref):
    p = part_ref[...]
    T = p[0] + p[1]
    hn_ref[...] = _agg_update(T, cnt_ref[...], h_ref[...], mW2_ref[...],
                              mb2_ref[...], uW_ref[...], ub_ref[...])


def _tail_body(n_real, h_ref, batch_ref, pat_ref,
               pW1_ref, pb1_ref, pW2_ref, pb2_ref, pW3_ref, pb3_ref,
               cW1_ref, cb1_ref, cW2_ref, cb2_ref, out_ref):
    NP = h_ref.shape[0]
    G = pat_ref.shape[0]
    bat = batch_ref[...]
    g = lax.broadcasted_iota(jnp.int32, (G, 1), 0)
    less = (bat < g).astype(jnp.float32)
    fi = jnp.sum(less, axis=1, keepdims=True)
    fi = jnp.minimum(fi, float(n_real - 1)).astype(jnp.int32)
    ii = lax.broadcasted_iota(jnp.int32, (G, NP), 1)
    onehot = (ii == fi).astype(jnp.float32)
    obs = jnp.dot(onehot, h_ref[...], preferred_element_type=jnp.float32)
    p = jnp.maximum(jnp.dot(pat_ref[...], pW1_ref[...],
                            preferred_element_type=jnp.float32) + pb1_ref[...], 0.0)
    p = jnp.maximum(jnp.dot(p, pW2_ref[...],
                            preferred_element_type=jnp.float32) + pb2_ref[...], 0.0)
    p = jnp.dot(p, pW3_ref[...], preferred_element_type=jnp.float32) + pb3_ref[...]
    z = jnp.concatenate([obs, p], axis=1)
    z = jnp.maximum(jnp.dot(z, cW1_ref[...],
                            preferred_element_type=jnp.float32) + cb1_ref[...], 0.0)
    out_ref[...] = jnp.dot(z, cW2_ref[...],
                           preferred_element_type=jnp.float32) + cb2_ref[...]


def _make_edge_sc(Nn, H, E, with_cnt):
    NW = _NC * _NS
    EW = E // NW
    K = _K
    NCH = EW // K
    RT = Nn // _NS
    ZR = 128
    W = H + _LN if with_cnt else H
    JH = H // _LN

    mesh = plsc.VectorSubcoreMesh(core_axis_name="c", subcore_axis_name="s")

    D = 4

    def body(A_hbm, src_hbm, dst_hbm, attr_hbm, v_hbm, out_hbm,
             src_v, attr_v, dst_vm, v_v,
             rows0, rows1, rows2, rows3, m0, m1, m2, m3, zb, S,
             gs0, gs1, gs2, gs3, ss0, ss1, ss2, ss3):
        rows = [rows0, rows1, rows2, rows3]
        ms = [m0, m1, m2, m3]
        gs = [gs0, gs1, gs2, gs3]
        ss = [ss0, ss1, ss2, ss3]
        cid = lax.axis_index("c")
        sid = lax.axis_index("s")
        wid = sid * _NC + cid
        base = wid * EW
        pltpu.sync_copy(src_hbm.at[pl.ds(base, EW)], src_v)
        pltpu.sync_copy(attr_hbm.at[pl.ds(base, EW)], attr_v)
        pltpu.sync_copy(dst_hbm.at[wid], dst_vm)
        pltpu.sync_copy(v_hbm, v_v)

        z16 = jnp.zeros((_LN,), jnp.float32)
        o16 = jnp.ones((_LN,), jnp.float32)

        def zrow(i, carry):
            for j in range(W // _LN):
                zb[i, pl.ds(j * _LN, _LN)] = z16
            return carry
        lax.fori_loop(0, ZR, zrow, 0)
        row0_ = sid * RT
        for kb in range(RT // ZR):
            pltpu.sync_copy(zb, S.at[pl.ds(row0_ + kb * ZR, ZR)])

        if with_cnt:
            def orow(i, carry):
                for m in ms:
                    m[i, pl.ds(H, _LN)] = o16
                return carry
            lax.fori_loop(0, K, orow, 0)
        plsc.subcore_barrier()

        vjs = [v_v[pl.ds(j * _LN, _LN)] for j in range(JH)]
        lane0 = jnp.zeros((_LN,), jnp.int32)

        def g_start(c, p):
            pltpu.async_copy(A_hbm.at[src_v.at[pl.ds(c * K, K)]],
                             rows[p], gs[p])

        def g_wait(c, p):
            pltpu.make_async_copy(
                A_hbm.at[src_v.at[pl.ds(c * K, K)]], rows[p], gs[p]).wait()

        def s_start(c, p):
            pltpu.async_copy(ms[p], S.at[dst_vm.at[c]], ss[p], add=True)

        def s_wait(c, p):
            pltpu.make_async_copy(ms[p], S.at[dst_vm.at[c]], ss[p]).wait()

        def compute(c, p):
            base16 = lane0 + c * K
            r, m = rows[p], ms[p]

            @plsc.parallel_loop(0, K, step=1, unroll=8)
            def edge(e):
                a16 = plsc.load_gather(attr_v, [base16 + e])
                for j in range(JH):
                    m[e, pl.ds(j * _LN, _LN)] = jnp.maximum(
                        r[e, pl.ds(j * _LN, _LN)] + a16 * vjs[j], 0.0)

        def do_chunk(c, p, first, prefetch):
            g_wait(c, p)
            if not first:
                s_wait(c, p)
            compute(c, p)
            s_start(c, p)
            if prefetch is not None:
                g_start(prefetch, p)

        last = NCH - 1
        for p in range(D):
            g_start(p, p)
        for p in range(D):
            do_chunk(jnp.int32(p), p, True, p + D)

        def block(i, carry):
            for p in range(D):
                c = D * i + p
                do_chunk(c, p, False, jnp.minimum(c + D, last))
            return carry
        lax.fori_loop(1, (NCH - 1) // D, block, 0)

        do_chunk(jnp.int32(last), 0, False, None)
        for p in range(1, D):
            g_wait(last, p)
            s_wait(last, p)
        s_wait(last, 0)

        plsc.subcore_barrier()
        for kb in range(RT // ZR):
            pltpu.sync_copy(S.at[pl.ds(row0_ + kb * ZR, ZR)],
                            out_hbm.at[cid, pl.ds(row0_ + kb * ZR, ZR)])

    return pl.kernel(
        body,
        out_type=jax.ShapeDtypeStruct((_NC, Nn, W), jnp.float32),
        mesh=mesh,
        compiler_params=pltpu.CompilerParams(needs_layout_passes=False,
                                             use_tc_tiling_on_sc=False),
        scratch_types=[
            pltpu.VMEM((EW,), jnp.int32),
            pltpu.VMEM((EW,), jnp.float32),
            pltpu.VMEM((NCH, K), jnp.int32),
            pltpu.VMEM((H,), jnp.float32),
            pltpu.VMEM((K, H), jnp.float32),
            pltpu.VMEM((K, H), jnp.float32),
            pltpu.VMEM((K, H), jnp.float32),
            pltpu.VMEM((K, H), jnp.float32),
            pltpu.VMEM((K, W), jnp.float32),
            pltpu.VMEM((K, W), jnp.float32),
            pltpu.VMEM((K, W), jnp.float32),
            pltpu.VMEM((K, W), jnp.float32),
            pltpu.VMEM((ZR, W), jnp.float32),
            pltpu.VMEM_SHARED((Nn, W), jnp.float32),
            pltpu.SemaphoreType.DMA,
            pltpu.SemaphoreType.DMA,
            pltpu.SemaphoreType.DMA,
            pltpu.SemaphoreType.DMA,
            pltpu.SemaphoreType.DMA,
            pltpu.SemaphoreType.DMA,
            pltpu.SemaphoreType.DMA,
            pltpu.SemaphoreType.DMA,
        ],
    )


def _make_cnt_sc(Nn, E):
    NW = _NC * _NS
    EW = E // NW
    mesh = plsc.VectorSubcoreMesh(core_axis_name="c", subcore_axis_name="s")

    def body(dst_hbm, out_hbm, dst_v, hist):
        cid = lax.axis_index("c")
        sid = lax.axis_index("s")
        wid = sid * _NC + cid
        pltpu.sync_copy(dst_hbm.at[pl.ds(wid * EW, EW)], dst_v)
        z16 = jnp.zeros((_LN,), jnp.float32)
        o16 = jnp.ones((_LN,), jnp.float32)

        def zrow(i, carry):
            hist[pl.ds(i * _LN, _LN)] = z16
            return carry
        lax.fori_loop(0, Nn // _LN, zrow, 0, unroll=8)

        def step(i, carry):
            idx = dst_v[pl.ds(i * _LN, _LN)]
            plsc.addupdate_scatter(hist, [idx], o16)
            return carry
        lax.fori_loop(0, EW // _LN, step, 0, unroll=8)
        pltpu.sync_copy(hist, out_hbm.at[cid, sid])

    return pl.kernel(
        body,
        out_type=jax.ShapeDtypeStruct((_NC, _NS, Nn), jnp.float32),
        mesh=mesh,
        compiler_params=pltpu.CompilerParams(needs_layout_passes=False,
                                             use_tc_tiling_on_sc=False),
        scratch_types=[
            pltpu.VMEM((EW,), jnp.int32),
            pltpu.VMEM((Nn,), jnp.float32),
        ],
    )


def kernel(x, edge_index, edge_attr, batch, mask,
           node_W, node_b, edge_W, edge_b,
           msg_W1_0, msg_b1_0, msg_W2_0, msg_b2_0, upd_W_0, upd_b_0,
           msg_W1_1, msg_b1_1, msg_W2_1, msg_b2_1, upd_W_1, upd_b_1,
           msg_W1_2, msg_b1_2, msg_W2_2, msg_b2_2, upd_W_2, upd_b_2,
           pat_W1, pat_b1, pat_W2, pat_b2, pat_W3, pat_b3,
           clf_W1, clf_b1, clf_W2, clf_b2):
    N, F = x.shape
    H = node_W.shape[1]
    E = edge_index.shape[1]
    G = mask.shape[0] // F
    C = clf_W2.shape[1]
    f32 = jnp.float32
    r1 = lambda b: b.reshape(1, -1)

    ALIGN = _NS * 128
    NP = ((N + ALIGN - 1) // ALIGN) * ALIGN
    src = edge_index[0]
    dst = edge_index[1].reshape(_NC * _NS, E // (_NC * _NS) // _K, _K)
    attr = edge_attr.reshape(E)
    xp = jnp.pad(x, ((0, NP - N), (0, 0)))
    batch_p = jnp.pad(batch, (0, NP - N), constant_values=jnp.int32(2**30))

    h, A, v = pl.pallas_call(
        _prep_body,
        out_shape=(jax.ShapeDtypeStruct((NP, H), f32),
                   jax.ShapeDtypeStruct((NP, H), f32),
                   jax.ShapeDtypeStruct((1, H), f32)),
    )(xp, node_W, r1(node_b), msg_W1_0, r1(msg_b1_0), edge_W, r1(edge_b))

    edge_sc = _make_edge_sc(NP, H, E, False)
    cnt_sc = _make_cnt_sc(NP, E)

    upd0 = pl.pallas_call(
        _upd0_body,
        out_shape=(jax.ShapeDtypeStruct((NP, H), f32),
                   jax.ShapeDtypeStruct((NP, H), f32),
                   jax.ShapeDtypeStruct((1, H), f32),
                   jax.ShapeDtypeStruct((NP, 1), f32)),
    )
    upd_mid = pl.pallas_call(
        _upd_mid_body,
        out_shape=(jax.ShapeDtypeStruct((NP, H), f32),
                   jax.ShapeDtypeStruct((NP, H), f32),
                   jax.ShapeDtypeStruct((1, H), f32)),
    )
    upd_last = pl.pallas_call(
        _upd_last_body,
        out_shape=jax.ShapeDtypeStruct((NP, H), f32),
    )

    cnthist = cnt_sc(edge_index[1])
    part = edge_sc(A, src, dst, attr, v.reshape(H))
    h, A, v, cnt = upd0(part, cnthist, h, msg_W2_0, r1(msg_b2_0),
                        upd_W_0, r1(upd_b_0),
                        msg_W1_1, r1(msg_b1_1), edge_W, r1(edge_b))
    part = edge_sc(A, src, dst, attr, v.reshape(H))
    h, A, v = upd_mid(part, cnt, h, msg_W2_1, r1(msg_b2_1),
                      upd_W_1, r1(upd_b_1),
                      msg_W1_2, r1(msg_b1_2), edge_W, r1(edge_b))
    part = edge_sc(A, src, dst, attr, v.reshape(H))
    h = upd_last(part, cnt, h, msg_W2_2, r1(msg_b2_2), upd_W_2, r1(upd_b_2))

    out = pl.pallas_call(
        functools.partial(_tail_body, N),
        out_shape=jax.ShapeDtypeStruct((G, C), f32),
    )(h, batch_p.reshape(1, NP), mask.reshape(G, F),
      pat_W1, r1(pat_b1), pat_W2, r1(pat_b2), pat_W3, r1(pat_b3),
      clf_W1, r1(clf_b1), clf_W2, r1(clf_b2))
    return out


# speedup vs baseline: 15.9018x; 1.0258x over previous
import functools

import jax
import jax.numpy as jnp
from jax import lax
from jax.experimental import pallas as pl
from jax.experimental.pallas import tpu as pltpu
from jax.experimental.pallas import tpu_sc as plsc

_NC = 2
_NS = 16
_LN = 16
_K = 80


def _prep_body(x_ref, nW_ref, nb_ref, mW1_ref, mb1_ref, eW_ref, eb_ref,
               h_ref, A_ref, v_ref):
    N = x_ref.shape[0]
    H = nW_ref.shape[1]
    h = jnp.dot(x_ref[...], nW_ref[...],
                preferred_element_type=jnp.float32) + nb_ref[...]
    W1a = mW1_ref[:H, :]
    W1b = mW1_ref[H:, :]
    c = jnp.dot(eb_ref[...], W1b,
                preferred_element_type=jnp.float32) + mb1_ref[...]
    h_ref[:N, :] = h
    A_ref[:N, :] = jnp.dot(h, W1a, preferred_element_type=jnp.float32) + c
    v_ref[...] = jnp.dot(eW_ref[...], W1b, preferred_element_type=jnp.float32)


def _agg_update(T, cnt, h, mW2, mb2, uW, ub):
    H = h.shape[1]
    seg = T / jnp.maximum(cnt, 1.0)
    aggr = jnp.dot(seg, mW2, preferred_element_type=jnp.float32) \
        + mb2 * (cnt > 0).astype(jnp.float32)
    hn = jnp.dot(h, uW[:H, :], preferred_element_type=jnp.float32) \
        + jnp.dot(aggr, uW[H:, :], preferred_element_type=jnp.float32) \
        + ub
    return jnp.maximum(hn, 0.0)


def _next_A(hn, mW1n, mb1n, eW, eb):
    H = hn.shape[1]
    W1a = mW1n[:H, :]
    W1b = mW1n[H:, :]
    c = jnp.dot(eb, W1b, preferred_element_type=jnp.float32) + mb1n
    An = jnp.dot(hn, W1a, preferred_element_type=jnp.float32) + c
    vn = jnp.dot(eW, W1b, preferred_element_type=jnp.float32)
    return An, vn


def _upd0_body(part_ref, ch_ref, h_ref, mW2_ref, mb2_ref, uW_ref, ub_ref,
               mW1n_ref, mb1n_ref, eW_ref, eb_ref,
               hn_ref, An_ref, vn_ref, cnt_ref):
    p = part_ref[...]
    T = p[0] + p[1]
    cnt = jnp.sum(ch_ref[...], axis=(0, 1))[:, None]
    cnt_ref[...] = cnt
    hn = _agg_update(T, cnt, h_ref[...], mW2_ref[...], mb2_ref[...],
                     uW_ref[...], ub_ref[...])
    hn_ref[...] = hn
    An_ref[...], vn_ref[...] = _next_A(hn, mW1n_ref[...], mb1n_ref[...],
                                       eW_ref[...], eb_ref[...])


def _upd_mid_body(part_ref, cnt_ref, h_ref, mW2_ref, mb2_ref, uW_ref, ub_ref,
                  mW1n_ref, mb1n_ref, eW_ref, eb_ref,
                  hn_ref, An_ref, vn_ref):
    p = part_ref[...]
    T = p[0] + p[1]
    hn = _agg_update(T, cnt_ref[...], h_ref[...], mW2_ref[...], mb2_ref[...],
                     uW_ref[...], ub_ref[...])
    hn_ref[...] = hn
    An_ref[...], vn_ref[...] = _next_A(hn, mW1n_ref[...], mb1n_ref[...],
                                       eW_ref[...], eb_ref[...])


def _tail_body(n_real, part_ref, cnt_ref, h_ref, mW2_ref, mb2_ref,
               uW_ref, ub_ref, batch_ref, pat_ref,
               pW1_ref, pb1_ref, pW2_ref, pb2_ref, pW3_ref, pb3_ref,
               cW1_ref, cb1_ref, cW2_ref, cb2_ref, out_ref):
    p = part_ref[...]
    T = p[0] + p[1]
    h = _agg_update(T, cnt_ref[...], h_ref[...], mW2_ref[...],
                    mb2_ref[...], uW_ref[...], ub_ref[...])
    NP = h.shape[0]
    G = pat_ref.shape[0]
    bat = batch_ref[...]
    g = lax.broadcasted_iota(jnp.int32, (G, 1), 0)
    less = (bat < g).astype(jnp.float32)
    fi = jnp.sum(less, axis=1, keepdims=True)
    fi = jnp.minimum(fi, float(n_real - 1)).astype(jnp.int32)
    ii = lax.broadcasted_iota(jnp.int32, (G, NP), 1)
    onehot = (ii == fi).astype(jnp.float32)
    obs = jnp.dot(onehot, h, preferred_element_type=jnp.float32)
    p = jnp.maximum(jnp.dot(pat_ref[...], pW1_ref[...],
                            preferred_element_type=jnp.float32) + pb1_ref[...], 0.0)
    p = jnp.maximum(jnp.dot(p, pW2_ref[...],
                            preferred_element_type=jnp.float32) + pb2_ref[...], 0.0)
    p = jnp.dot(p, pW3_ref[...], preferred_element_type=jnp.float32) + pb3_ref[...]
    z = jnp.concatenate([obs, p], axis=1)
    z = jnp.maximum(jnp.dot(z, cW1_ref[...],
                            preferred_element_type=jnp.float32) + cb1_ref[...], 0.0)
    out_ref[...] = jnp.dot(z, cW2_ref[...],
                           preferred_element_type=jnp.float32) + cb2_ref[...]


def _make_edge_sc(Nn, H, E, with_cnt):
    NW = _NC * _NS
    EW = E // NW
    K = _K
    NCH = EW // K
    RT = Nn // _NS
    ZR = 128
    W = H + _LN if with_cnt else H
    JH = H // _LN

    mesh = plsc.VectorSubcoreMesh(core_axis_name="c", subcore_axis_name="s")

    D = 4

    def body(A_hbm, src_hbm, dst_hbm, attr_hbm, v_hbm, out_hbm,
             src_v, attr_v, dst_vm, v_v,
             rows0, rows1, rows2, rows3, m0, m1, m2, m3, zb, S,
             gs0, gs1, gs2, gs3, ss0, ss1, ss2, ss3):
        rows = [rows0, rows1, rows2, rows3]
        ms = [m0, m1, m2, m3]
        gs = [gs0, gs1, gs2, gs3]
        ss = [ss0, ss1, ss2, ss3]
        cid = lax.axis_index("c")
        sid = lax.axis_index("s")
        wid = sid * _NC + cid
        base = wid * EW
        pltpu.sync_copy(src_hbm.at[pl.ds(base, EW)], src_v)
        pltpu.sync_copy(attr_hbm.at[pl.ds(base, EW)], attr_v)
        pltpu.sync_copy(dst_hbm.at[wid], dst_vm)
        pltpu.sync_copy(v_hbm, v_v)

        z16 = jnp.zeros((_LN,), jnp.float32)
        o16 = jnp.ones((_LN,), jnp.float32)

        def zrow(i, carry):
            for j in range(W // _LN):
                zb[i, pl.ds(j * _LN, _LN)] = z16
            return carry
        lax.fori_loop(0, ZR, zrow, 0)
        row0_ = sid * RT
        for kb in range(RT // ZR):
            pltpu.sync_copy(zb, S.at[pl.ds(row0_ + kb * ZR, ZR)])

        if with_cnt:
            def orow(i, carry):
                for m in ms:
                    m[i, pl.ds(H, _LN)] = o16
                return carry
            lax.fori_loop(0, K, orow, 0)
        plsc.subcore_barrier()

        vjs = [v_v[pl.ds(j * _LN, _LN)] for j in range(JH)]
        lane0 = jnp.zeros((_LN,), jnp.int32)

        def g_start(c, p):
            pltpu.async_copy(A_hbm.at[src_v.at[pl.ds(c * K, K)]],
                             rows[p], gs[p])

        def g_wait(c, p):
            pltpu.make_async_copy(
                A_hbm.at[src_v.at[pl.ds(c * K, K)]], rows[p], gs[p]).wait()

        def s_start(c, p):
            pltpu.async_copy(ms[p], S.at[dst_vm.at[c]], ss[p], add=True)

        def s_wait(c, p):
            pltpu.make_async_copy(ms[p], S.at[dst_vm.at[c]], ss[p]).wait()

        def compute(c, p):
            base16 = lane0 + c * K
            r, m = rows[p], ms[p]

            @plsc.parallel_loop(0, K, step=1, unroll=8)
            def edge(e):
                a16 = plsc.load_gather(attr_v, [base16 + e])
                for j in range(JH):
                    m[e, pl.ds(j * _LN, _LN)] = jnp.maximum(
                        r[e, pl.ds(j * _LN, _LN)] + a16 * vjs[j], 0.0)

        def do_chunk(c, p, first, prefetch):
            g_wait(c, p)
            if not first:
                s_wait(c, p)
            compute(c, p)
            s_start(c, p)
            if prefetch is not None:
                g_start(prefetch, p)

        last = NCH - 1
        for p in range(D):
            g_start(p, p)
        for p in range(D):
            do_chunk(jnp.int32(p), p, True, p + D)

        def block(i, carry):
            for p in range(D):
                c = D * i + p
                do_chunk(c, p, False, jnp.minimum(c + D, last))
            return carry
        lax.fori_loop(1, (NCH - 1) // D, block, 0)

        do_chunk(jnp.int32(last), 0, False, None)
        for p in range(1, D):
            g_wait(last, p)
            s_wait(last, p)
        s_wait(last, 0)

        plsc.subcore_barrier()
        for kb in range(RT // ZR):
            pltpu.sync_copy(S.at[pl.ds(row0_ + kb * ZR, ZR)],
                            out_hbm.at[cid, pl.ds(row0_ + kb * ZR, ZR)])

    return pl.kernel(
        body,
        out_type=jax.ShapeDtypeStruct((_NC, Nn, W), jnp.float32),
        mesh=mesh,
        compiler_params=pltpu.CompilerParams(needs_layout_passes=False,
                                             use_tc_tiling_on_sc=False),
        scratch_types=[
            pltpu.VMEM((EW,), jnp.int32),
            pltpu.VMEM((EW,), jnp.float32),
            pltpu.VMEM((NCH, K), jnp.int32),
            pltpu.VMEM((H,), jnp.float32),
            pltpu.VMEM((K, H), jnp.float32),
            pltpu.VMEM((K, H), jnp.float32),
            pltpu.VMEM((K, H), jnp.float32),
            pltpu.VMEM((K, H), jnp.float32),
            pltpu.VMEM((K, W), jnp.float32),
            pltpu.VMEM((K, W), jnp.float32),
            pltpu.VMEM((K, W), jnp.float32),
            pltpu.VMEM((K, W), jnp.float32),
            pltpu.VMEM((ZR, W), jnp.float32),
            pltpu.VMEM_SHARED((Nn, W), jnp.float32),
            pltpu.SemaphoreType.DMA,
            pltpu.SemaphoreType.DMA,
            pltpu.SemaphoreType.DMA,
            pltpu.SemaphoreType.DMA,
            pltpu.SemaphoreType.DMA,
            pltpu.SemaphoreType.DMA,
            pltpu.SemaphoreType.DMA,
            pltpu.SemaphoreType.DMA,
        ],
    )


def _make_cnt_sc(Nn, E):
    NW = _NC * _NS
    EW = E // NW
    mesh = plsc.VectorSubcoreMesh(core_axis_name="c", subcore_axis_name="s")

    def body(dst_hbm, out_hbm, dst_v, hist):
        cid = lax.axis_index("c")
        sid = lax.axis_index("s")
        wid = sid * _NC + cid
        pltpu.sync_copy(dst_hbm.at[pl.ds(wid * EW, EW)], dst_v)
        z16 = jnp.zeros((_LN,), jnp.float32)
        o16 = jnp.ones((_LN,), jnp.float32)

        def zrow(i, carry):
            hist[pl.ds(i * _LN, _LN)] = z16
            return carry
        lax.fori_loop(0, Nn // _LN, zrow, 0, unroll=8)

        def step(i, carry):
            idx = dst_v[pl.ds(i * _LN, _LN)]
            plsc.addupdate_scatter(hist, [idx], o16)
            return carry
        lax.fori_loop(0, EW // _LN, step, 0, unroll=8)
        pltpu.sync_copy(hist, out_hbm.at[cid, sid])

    return pl.kernel(
        body,
        out_type=jax.ShapeDtypeStruct((_NC, _NS, Nn), jnp.float32),
        mesh=mesh,
        compiler_params=pltpu.CompilerParams(needs_layout_passes=False,
                                             use_tc_tiling_on_sc=False),
        scratch_types=[
            pltpu.VMEM((EW,), jnp.int32),
            pltpu.VMEM((Nn,), jnp.float32),
        ],
    )


def kernel(x, edge_index, edge_attr, batch, mask,
           node_W, node_b, edge_W, edge_b,
           msg_W1_0, msg_b1_0, msg_W2_0, msg_b2_0, upd_W_0, upd_b_0,
           msg_W1_1, msg_b1_1, msg_W2_1, msg_b2_1, upd_W_1, upd_b_1,
           msg_W1_2, msg_b1_2, msg_W2_2, msg_b2_2, upd_W_2, upd_b_2,
           pat_W1, pat_b1, pat_W2, pat_b2, pat_W3, pat_b3,
           clf_W1, clf_b1, clf_W2, clf_b2):
    N, F = x.shape
    H = node_W.shape[1]
    E = edge_index.shape[1]
    G = mask.shape[0] // F
    C = clf_W2.shape[1]
    f32 = jnp.float32
    r1 = lambda b: b.reshape(1, -1)

    ALIGN = _NS * 128
    NP = ((N + ALIGN - 1) // ALIGN) * ALIGN
    src = edge_index[0]
    dst = edge_index[1].reshape(_NC * _NS, E // (_NC * _NS) // _K, _K)
    attr = edge_attr.reshape(E)
    batch_p = jnp.pad(batch, (0, NP - N), constant_values=jnp.int32(2**30))

    h, A, v = pl.pallas_call(
        _prep_body,
        out_shape=(jax.ShapeDtypeStruct((NP, H), f32),
                   jax.ShapeDtypeStruct((NP, H), f32),
                   jax.ShapeDtypeStruct((1, H), f32)),
    )(x, node_W, r1(node_b), msg_W1_0, r1(msg_b1_0), edge_W, r1(edge_b))

    edge_sc = _make_edge_sc(NP, H, E, False)
    cnt_sc = _make_cnt_sc(NP, E)

    upd0 = pl.pallas_call(
        _upd0_body,
        out_shape=(jax.ShapeDtypeStruct((NP, H), f32),
                   jax.ShapeDtypeStruct((NP, H), f32),
                   jax.ShapeDtypeStruct((1, H), f32),
                   jax.ShapeDtypeStruct((NP, 1), f32)),
    )
    upd_mid = pl.pallas_call(
        _upd_mid_body,
        out_shape=(jax.ShapeDtypeStruct((NP, H), f32),
                   jax.ShapeDtypeStruct((NP, H), f32),
                   jax.ShapeDtypeStruct((1, H), f32)),
    )
    cnthist = cnt_sc(edge_index[1])
    part = edge_sc(A, src, dst, attr, v.reshape(H))
    h, A, v, cnt = upd0(part, cnthist, h, msg_W2_0, r1(msg_b2_0),
                        upd_W_0, r1(upd_b_0),
                        msg_W1_1, r1(msg_b1_1), edge_W, r1(edge_b))
    part = edge_sc(A, src, dst, attr, v.reshape(H))
    h, A, v = upd_mid(part, cnt, h, msg_W2_1, r1(msg_b2_1),
                      upd_W_1, r1(upd_b_1),
                      msg_W1_2, r1(msg_b1_2), edge_W, r1(edge_b))
    part = edge_sc(A, src, dst, attr, v.reshape(H))

    out = pl.pallas_call(
        functools.partial(_tail_body, N),
        out_shape=jax.ShapeDtypeStruct((G, C), f32),
    )(part, cnt, h, msg_W2_2, r1(msg_b2_2), upd_W_2, r1(upd_b_2),
      batch_p.reshape(1, NP), mask.reshape(G, F),
      pat_W1, r1(pat_b1), pat_W2, r1(pat_b2), pat_W3, r1(pat_b3),
      clf_W1, r1(clf_b1), clf_W2, r1(clf_b2))
    return out
